# Initial kernel scaffold; baseline (speedup 1.0000x reference)
import functools

import jax
import jax.numpy as jnp
from jax import lax
from jax.experimental import pallas as pl
from jax.experimental.pallas import tpu as pltpu
from jax.experimental.pallas import tpu_sc as plsc

N = 10000
E = 160000
EPS = 1e-5
HSTEP = 0.1

NC = 2
NS = 16
NW = NC * NS
EPW = E // NW
CH = 125
NCH = EPW // CH
NPS = N // NS

EB = 2000
NEB = E // EB

_f32 = jnp.float32


def _mesh():
    return plsc.VectorSubcoreMesh(core_axis_name="c", subcore_axis_name="s",
                                  num_cores=NC, num_subcores=NS)


def _sc_gather_body(table_hbm, idx_hbm, out_hbm, idx_v, rows_v, gsem):
    cid = lax.axis_index("c")
    sid = lax.axis_index("s")
    wid = sid * NC + cid
    base = wid * EPW
    pltpu.sync_copy(idx_hbm.at[wid], idx_v)

    def chunk(c, carry):
        pltpu.async_copy(table_hbm.at[idx_v.at[c]], rows_v, gsem).wait()
        pltpu.sync_copy(rows_v, out_hbm.at[pl.ds(base + c * CH, CH)])
        return carry

    lax.fori_loop(0, NCH, chunk, 0)


def _sc_gather(table, idx3):
    return pl.kernel(
        _sc_gather_body,
        out_type=jax.ShapeDtypeStruct((E, 64), _f32),
        mesh=_mesh(),
        scratch_types=[
            pltpu.VMEM((NCH, CH), jnp.int32),
            pltpu.VMEM((CH, 64), _f32),
            pltpu.SemaphoreType.DMA,
        ],
    )(table, idx3)


def _sc_scatter_body(g_hbm, ii_hbm, jj_hbm, z_hbm, out_hbm,
                     ii_v, jj_v, rows_v, acc_i, acc_j):
    cid = lax.axis_index("c")
    sid = lax.axis_index("s")
    wid = sid * NC + cid
    base = wid * EPW
    pltpu.sync_copy(z_hbm.at[pl.ds(sid * NPS, NPS)],
                    acc_i.at[pl.ds(sid * NPS, NPS)])
    pltpu.sync_copy(z_hbm.at[pl.ds(sid * NPS, NPS)],
                    acc_j.at[pl.ds(sid * NPS, NPS)])
    pltpu.sync_copy(ii_hbm.at[wid], ii_v)
    pltpu.sync_copy(jj_hbm.at[wid], jj_v)
    plsc.subcore_barrier()

    def chunk(c, carry):
        pltpu.sync_copy(g_hbm.at[pl.ds(base + c * CH, CH)], rows_v)
        pltpu.sync_copy(rows_v, acc_i.at[ii_v.at[c]], add=True)
        pltpu.sync_copy(rows_v, acc_j.at[jj_v.at[c]], add=True)
        return carry

    lax.fori_loop(0, NCH, chunk, 0)
    plsc.subcore_barrier()
    sl = pl.ds(sid * NPS, NPS)
    pltpu.sync_copy(acc_i.at[sl], out_hbm.at[cid, 0, sl])
    pltpu.sync_copy(acc_j.at[sl], out_hbm.at[cid, 1, sl])


def _sc_scatter(g, ii3, jj3, zeros_n):
    return pl.kernel(
        _sc_scatter_body,
        out_type=jax.ShapeDtypeStruct((NC, 2, N, 64), _f32),
        mesh=_mesh(),
        scratch_types=[
            pltpu.VMEM((NCH, CH), jnp.int32),
            pltpu.VMEM((NCH, CH), jnp.int32),
            pltpu.VMEM((CH, 64), _f32),
            pltpu.VMEM_SHARED((N, 64), _f32),
            pltpu.VMEM_SHARED((N, 64), _f32),
        ],
    )(g, ii3, jj3, zeros_n)


def _ln_relu(h):
    cnt = float(h.shape[0] * h.shape[1])
    m = jnp.sum(h) / cnt
    v = jnp.sum((h - m) * (h - m)) / cnt
    return jnp.maximum((h - m) * lax.rsqrt(v + EPS), 0.0)


def _node_open_body(xn_ref, w1t_ref, w2t_ref, out_ref):
    h = jnp.dot(xn_ref[...], w1t_ref[...], preferred_element_type=_f32)
    h = _ln_relu(h)
    out_ref[...] = jnp.dot(h, w2t_ref[...], preferred_element_type=_f32)


def _node_open(xn_t, w1t, w2t):
    return pl.pallas_call(
        _node_open_body,
        out_shape=jax.ShapeDtypeStruct((N, 64), _f32),
    )(xn_t, w1t, w2t)


def _node_layer_body(parts_ref, xn_ref, wat_ref, wbt_ref, wct_ref, w2t_ref,
                     out_ref):
    p = parts_ref[...]
    s_i = p[0, 0] + p[1, 0]
    s_j = p[0, 1] + p[1, 1]
    ave = 0.5 * (s_i + s_j)
    div = s_i - s_j
    xn = xn_ref[...]
    h = jnp.dot(ave, wat_ref[...], preferred_element_type=_f32)
    h = h + jnp.dot(div, wbt_ref[...], preferred_element_type=_f32)
    h = h + jnp.dot(xn, wct_ref[...], preferred_element_type=_f32)
    h = _ln_relu(h)
    out_ref[...] = xn + HSTEP * jnp.dot(h, w2t_ref[...],
                                        preferred_element_type=_f32)


def _node_layer(parts, xn, wat, wbt, wct, w2t):
    return pl.pallas_call(
        _node_layer_body,
        out_shape=jax.ShapeDtypeStruct((N, 64), _f32),
    )(parts, xn, wat, wbt, wct, w2t)


def _final_body(xn_ref, wt_ref, out_ref):
    out_ref[...] = jnp.dot(xn_ref[...], wt_ref[...],
                           preferred_element_type=_f32)


def _final(xn, wt):
    return pl.pallas_call(
        _final_body,
        out_shape=jax.ShapeDtypeStruct((N, 128), _f32),
    )(xn, wt)


def _wspec(shape):
    return pl.BlockSpec(shape, lambda i: (0, 0))


def _eopen_stats_body(xe_ref, w1t_ref, sum_ref, sq_ref):
    i = pl.program_id(0)
    h = jnp.dot(xe_ref[...], w1t_ref[...], preferred_element_type=_f32)

    @pl.when(i == 0)
    def _():
        sum_ref[...] = jnp.zeros_like(sum_ref)
        sq_ref[...] = jnp.zeros_like(sq_ref)

    sum_ref[...] += jnp.sum(h, axis=0, keepdims=True)
    sq_ref[...] += jnp.sum(h * h, axis=0, keepdims=True)


def _eopen_stats(xe_t, w1t):
    return pl.pallas_call(
        _eopen_stats_body,
        grid=(NEB,),
        in_specs=[pl.BlockSpec((EB, 16), lambda i: (i, 0)), _wspec((16, 64))],
        out_specs=[pl.BlockSpec((1, 64), lambda i: (0, 0)),
                   pl.BlockSpec((1, 64), lambda i: (0, 0))],
        out_shape=[jax.ShapeDtypeStruct((1, 64), _f32),
                   jax.ShapeDtypeStruct((1, 64), _f32)],
    )(xe_t, w1t)


def _eopen_apply_body(xe_ref, w1t_ref, w2t_ref, sum_ref, sq_ref, out_ref):
    cnt = float(E * 64)
    m = jnp.sum(sum_ref[...]) / cnt
    v = jnp.sum(sq_ref[...]) / cnt - m * m
    h = jnp.dot(xe_ref[...], w1t_ref[...], preferred_element_type=_f32)
    h = jnp.maximum((h - m) * lax.rsqrt(v + EPS), 0.0)
    out_ref[...] = jnp.dot(h, w2t_ref[...], preferred_element_type=_f32)


def _eopen_apply(xe_t, w1t, w2t, sumv, sqv):
    return pl.pallas_call(
        _eopen_apply_body,
        grid=(NEB,),
        in_specs=[pl.BlockSpec((EB, 16), lambda i: (i, 0)),
                  _wspec((16, 64)), _wspec((64, 64)),
                  _wspec((1, 64)), _wspec((1, 64))],
        out_specs=pl.BlockSpec((EB, 64), lambda i: (i, 0)),
        out_shape=jax.ShapeDtypeStruct((E, 64), _f32),
    )(xe_t, w1t, w2t, sumv, sqv)


def _estats_body(xg_ref, xe_ref, wat_ref, wbt_ref, sum_ref, sq_ref):
    i = pl.program_id(0)
    h = jnp.dot(xg_ref[...], wat_ref[...], preferred_element_type=_f32)
    h = h + jnp.dot(xe_ref[...], wbt_ref[...], preferred_element_type=_f32)

    @pl.when(i == 0)
    def _():
        sum_ref[...] = jnp.zeros_like(sum_ref)
        sq_ref[...] = jnp.zeros_like(sq_ref)

    sum_ref[...] += jnp.sum(h, axis=0, keepdims=True)
    sq_ref[...] += jnp.sum(h * h, axis=0, keepdims=True)


def _estats(xg, xe, wat, wbt):
    return pl.pallas_call(
        _estats_body,
        grid=(NEB,),
        in_specs=[pl.BlockSpec((EB, 64), lambda i: (i, 0)),
                  pl.BlockSpec((EB, 64), lambda i: (i, 0)),
                  _wspec((64, 128)), _wspec((64, 128))],
        out_specs=[pl.BlockSpec((1, 128), lambda i: (0, 0)),
                   pl.BlockSpec((1, 128), lambda i: (0, 0))],
        out_shape=[jax.ShapeDtypeStruct((1, 128), _f32),
                   jax.ShapeDtypeStruct((1, 128), _f32)],
    )(xg, xe, wat, wbt)


def _eapply_body(xg_ref, xe_ref, wat_ref, wbt_ref, w2t_ref, sum_ref, sq_ref,
                 g_ref, xeo_ref):
    cnt = float(E * 128)
    m = jnp.sum(sum_ref[...]) / cnt
    v = jnp.sum(sq_ref[...]) / cnt - m * m
    h = jnp.dot(xg_ref[...], wat_ref[...], preferred_element_type=_f32)
    h = h + jnp.dot(xe_ref[...], wbt_ref[...], preferred_element_type=_f32)
    h = jnp.maximum((h - m) * lax.rsqrt(v + EPS), 0.0)
    g = jnp.dot(h, w2t_ref[...], preferred_element_type=_f32)
    g_ref[...] = g
    xeo_ref[...] = xe_ref[...] + HSTEP * g


def _eapply(xg, xe, wat, wbt, w2t, sumv, sqv):
    return pl.pallas_call(
        _eapply_body,
        grid=(NEB,),
        in_specs=[pl.BlockSpec((EB, 64), lambda i: (i, 0)),
                  pl.BlockSpec((EB, 64), lambda i: (i, 0)),
                  _wspec((64, 128)), _wspec((64, 128)), _wspec((128, 64)),
                  _wspec((1, 128)), _wspec((1, 128))],
        out_specs=[pl.BlockSpec((EB, 64), lambda i: (i, 0)),
                   pl.BlockSpec((EB, 64), lambda i: (i, 0))],
        out_shape=[jax.ShapeDtypeStruct((E, 64), _f32),
                   jax.ShapeDtypeStruct((E, 64), _f32)],
    )(xg, xe, wat, wbt, w2t, sumv, sqv)


def kernel(xn, xe, iInd, jInd, K1Nopen, K2Nopen, K1Eopen, K2Eopen, KNout,
           KE1, KE2, KN1, KN2):
    xn_t = jnp.transpose(xn[0])
    xe_t = jnp.transpose(xe[0])
    ii3 = iInd.astype(jnp.int32).reshape(NW, NCH, CH)
    jj3 = jInd.astype(jnp.int32).reshape(NW, NCH, CH)
    zeros_n = jnp.zeros((N, 64), _f32)

    xn_c = _node_open(xn_t, K1Nopen.T, K2Nopen.T)
    sumv, sqv = _eopen_stats(xe_t, K1Eopen.T)
    xe_c = _eopen_apply(xe_t, K1Eopen.T, K2Eopen.T, sumv, sqv)

    nlayers = KE1.shape[0]
    for i in range(nlayers):
        wat = jnp.transpose(KE1[i][:, :64] + KE1[i][:, 64:128])
        wbt = jnp.transpose(KE1[i][:, 128:])
        w2t = jnp.transpose(KE2[i])
        xg = _sc_gather(xn_c, ii3)
        sumv, sqv = _estats(xg, xe_c, wat, wbt)
        g, xe_c = _eapply(xg, xe_c, wat, wbt, w2t, sumv, sqv)
        parts = _sc_scatter(g, ii3, jj3, zeros_n)
        xn_c = _node_layer(parts, xn_c,
                           jnp.transpose(KN1[i][:, :64]),
                           jnp.transpose(KN1[i][:, 64:128]),
                           jnp.transpose(KN1[i][:, 128:]),
                           jnp.transpose(KN2[i]))

    xn_out = _final(xn_c, KNout.T)
    return jnp.transpose(xn_out)[None], jnp.transpose(xe_c)[None]


# speedup vs baseline: 2.1789x; 2.1789x over previous
import jax
import jax.numpy as jnp
from jax import lax
from jax.experimental import pallas as pl
from jax.experimental.pallas import tpu as pltpu
from jax.experimental.pallas import tpu_sc as plsc

N = 10000
E = 160000
EPS = 1e-5
HSTEP = 0.1

NC = 2
NS = 16
NW = NC * NS
EPW = E // NW
CH = 40
NCH = EPW // CH
NPAD = 10240
NPS = NPAD // NS

EB = 2000
NEB = E // EB

_f32 = jnp.float32


def _mesh():
    return plsc.VectorSubcoreMesh(core_axis_name="c", subcore_axis_name="s",
                                  num_cores=NC, num_subcores=NS)


def _sc_gather_body(table_hbm, idx_hbm, out_hbm, idx_v, rows_v, gsem):
    cid = lax.axis_index("c")
    sid = lax.axis_index("s")
    wid = sid * NC + cid
    base = wid * EPW
    pltpu.sync_copy(idx_hbm.at[wid], idx_v)

    def chunk(c, carry):
        pltpu.async_copy(table_hbm.at[idx_v.at[c]], rows_v, gsem).wait()
        pltpu.sync_copy(rows_v, out_hbm.at[pl.ds(base + c * CH, CH)])
        return carry

    lax.fori_loop(0, NCH, chunk, 0)


def _sc_gather(table, idx3):
    return pl.kernel(
        _sc_gather_body,
        out_type=jax.ShapeDtypeStruct((E, 128), _f32),
        mesh=_mesh(),
        scratch_types=[
            pltpu.VMEM((NCH, CH), jnp.int32),
            pltpu.VMEM((CH, 128), _f32),
            pltpu.SemaphoreType.DMA,
        ],
    )(table, idx3)


def _sc_scatter_body(g_hbm, ii_hbm, jj_hbm, z_hbm, out_hbm,
                     ii_v, jj_v, rows_v, acc):
    cid = lax.axis_index("c")
    sid = lax.axis_index("s")
    wid = sid * NC + cid
    base = wid * EPW
    sl = pl.ds(sid * NPS, NPS)
    pltpu.sync_copy(ii_hbm.at[wid], ii_v)
    pltpu.sync_copy(jj_hbm.at[wid], jj_v)
    pltpu.sync_copy(z_hbm.at[sl], acc.at[sl])
    plsc.subcore_barrier()

    def chunk_i(c, carry):
        pltpu.sync_copy(g_hbm.at[pl.ds(base + c * CH, CH)], rows_v)
        pltpu.sync_copy(rows_v, acc.at[ii_v.at[c]], add=True)
        return carry

    lax.fori_loop(0, NCH, chunk_i, 0)
    plsc.subcore_barrier()
    pltpu.sync_copy(acc.at[sl], out_hbm.at[cid, 0, sl])
    pltpu.sync_copy(z_hbm.at[sl], acc.at[sl])
    plsc.subcore_barrier()

    def chunk_j(c, carry):
        pltpu.sync_copy(g_hbm.at[pl.ds(base + c * CH, CH)], rows_v)
        pltpu.sync_copy(rows_v, acc.at[jj_v.at[c]], add=True)
        return carry

    lax.fori_loop(0, NCH, chunk_j, 0)
    plsc.subcore_barrier()
    pltpu.sync_copy(acc.at[sl], out_hbm.at[cid, 1, sl])


def _sc_scatter(g, ii3, jj3, zeros_n):
    return pl.kernel(
        _sc_scatter_body,
        out_type=jax.ShapeDtypeStruct((NC, 2, NPAD, 128), _f32),
        mesh=_mesh(),
        scratch_types=[
            pltpu.VMEM((NCH, CH), jnp.int32),
            pltpu.VMEM((NCH, CH), jnp.int32),
            pltpu.VMEM((CH, 128), _f32),
            pltpu.VMEM_SHARED((NPAD, 128), _f32),
        ],
    )(g, ii3, jj3, zeros_n)


def _ln_relu(h):
    cnt = float(h.shape[0] * h.shape[1])
    m = jnp.sum(h) / cnt
    v = jnp.sum((h - m) * (h - m)) / cnt
    return jnp.maximum((h - m) * lax.rsqrt(v + EPS), 0.0)


def _node_open_body(xn_ref, w1t_ref, w2t_ref, out_ref):
    h = jnp.dot(xn_ref[...], w1t_ref[...], preferred_element_type=_f32)
    h = _ln_relu(h)
    r = jnp.dot(h, w2t_ref[...], preferred_element_type=_f32)
    out_ref[...] = jnp.concatenate([r, jnp.zeros_like(r)], axis=1)


def _node_open(xn_t, w1t, w2t):
    return pl.pallas_call(
        _node_open_body,
        out_shape=jax.ShapeDtypeStruct((N, 128), _f32),
    )(xn_t, w1t, w2t)


def _node_layer_body(parts_ref, xn_ref, wat_ref, wbt_ref, wct_ref, w2t_ref,
                     out_ref):
    p = parts_ref[...]
    s_i = p[0, 0, :N, :64] + p[1, 0, :N, :64]
    s_j = p[0, 1, :N, :64] + p[1, 1, :N, :64]
    ave = 0.5 * (s_i + s_j)
    div = s_i - s_j
    xn = xn_ref[...]
    h = jnp.dot(ave, wat_ref[...], preferred_element_type=_f32)
    h = h + jnp.dot(div, wbt_ref[...], preferred_element_type=_f32)
    h = h + jnp.dot(xn, wct_ref[...], preferred_element_type=_f32)
    h = _ln_relu(h)
    r = jnp.dot(h, w2t_ref[...], preferred_element_type=_f32)
    out_ref[...] = xn + HSTEP * jnp.concatenate(
        [r, jnp.zeros_like(r)], axis=1)


def _node_layer(parts, xn, wat, wbt, wct, w2t):
    return pl.pallas_call(
        _node_layer_body,
        out_shape=jax.ShapeDtypeStruct((N, 128), _f32),
    )(parts, xn, wat, wbt, wct, w2t)


def _final_body(xn_ref, wt_ref, out_ref):
    out_ref[...] = jnp.dot(xn_ref[...], wt_ref[...],
                           preferred_element_type=_f32)


def _final(xn, wt):
    return pl.pallas_call(
        _final_body,
        out_shape=jax.ShapeDtypeStruct((N, 128), _f32),
    )(xn, wt)


def _wspec(shape):
    return pl.BlockSpec(shape, lambda i: (0, 0))


def _eopen_stats_body(xe_ref, w1t_ref, sum_ref, sq_ref):
    i = pl.program_id(0)
    h = jnp.dot(xe_ref[...], w1t_ref[...], preferred_element_type=_f32)

    @pl.when(i == 0)
    def _():
        sum_ref[...] = jnp.zeros_like(sum_ref)
        sq_ref[...] = jnp.zeros_like(sq_ref)

    sum_ref[...] += jnp.sum(h, axis=0, keepdims=True)
    sq_ref[...] += jnp.sum(h * h, axis=0, keepdims=True)


def _eopen_stats(xe_t, w1t):
    return pl.pallas_call(
        _eopen_stats_body,
        grid=(NEB,),
        in_specs=[pl.BlockSpec((EB, 16), lambda i: (i, 0)), _wspec((16, 64))],
        out_specs=[pl.BlockSpec((1, 64), lambda i: (0, 0)),
                   pl.BlockSpec((1, 64), lambda i: (0, 0))],
        out_shape=[jax.ShapeDtypeStruct((1, 64), _f32),
                   jax.ShapeDtypeStruct((1, 64), _f32)],
    )(xe_t, w1t)


def _eopen_apply_body(xe_ref, w1t_ref, w2t_ref, sum_ref, sq_ref, out_ref):
    cnt = float(E * 64)
    m = jnp.sum(sum_ref[...]) / cnt
    v = jnp.sum(sq_ref[...]) / cnt - m * m
    h = jnp.dot(xe_ref[...], w1t_ref[...], preferred_element_type=_f32)
    h = jnp.maximum((h - m) * lax.rsqrt(v + EPS), 0.0)
    out_ref[...] = jnp.dot(h, w2t_ref[...], preferred_element_type=_f32)


def _eopen_apply(xe_t, w1t, w2t, sumv, sqv):
    return pl.pallas_call(
        _eopen_apply_body,
        grid=(NEB,),
        in_specs=[pl.BlockSpec((EB, 16), lambda i: (i, 0)),
                  _wspec((16, 64)), _wspec((64, 64)),
                  _wspec((1, 64)), _wspec((1, 64))],
        out_specs=pl.BlockSpec((EB, 64), lambda i: (i, 0)),
        out_shape=jax.ShapeDtypeStruct((E, 64), _f32),
    )(xe_t, w1t, w2t, sumv, sqv)


def _estats_body(xg_ref, xe_ref, wat_ref, wbt_ref, sum_ref, sq_ref):
    i = pl.program_id(0)
    h = jnp.dot(xg_ref[...], wat_ref[...], preferred_element_type=_f32)
    h = h + jnp.dot(xe_ref[...], wbt_ref[...], preferred_element_type=_f32)

    @pl.when(i == 0)
    def _():
        sum_ref[...] = jnp.zeros_like(sum_ref)
        sq_ref[...] = jnp.zeros_like(sq_ref)

    sum_ref[...] += jnp.sum(h, axis=0, keepdims=True)
    sq_ref[...] += jnp.sum(h * h, axis=0, keepdims=True)


def _estats(xg, xe, wat, wbt):
    return pl.pallas_call(
        _estats_body,
        grid=(NEB,),
        in_specs=[pl.BlockSpec((EB, 128), lambda i: (i, 0)),
                  pl.BlockSpec((EB, 64), lambda i: (i, 0)),
                  _wspec((128, 128)), _wspec((64, 128))],
        out_specs=[pl.BlockSpec((1, 128), lambda i: (0, 0)),
                   pl.BlockSpec((1, 128), lambda i: (0, 0))],
        out_shape=[jax.ShapeDtypeStruct((1, 128), _f32),
                   jax.ShapeDtypeStruct((1, 128), _f32)],
    )(xg, xe, wat, wbt)


def _eapply_body(xg_ref, xe_ref, wat_ref, wbt_ref, w2t_ref, sum_ref, sq_ref,
                 g_ref, xeo_ref):
    cnt = float(E * 128)
    m = jnp.sum(sum_ref[...]) / cnt
    v = jnp.sum(sq_ref[...]) / cnt - m * m
    h = jnp.dot(xg_ref[...], wat_ref[...], preferred_element_type=_f32)
    h = h + jnp.dot(xe_ref[...], wbt_ref[...], preferred_element_type=_f32)
    h = jnp.maximum((h - m) * lax.rsqrt(v + EPS), 0.0)
    g = jnp.dot(h, w2t_ref[...], preferred_element_type=_f32)
    g_ref[...] = jnp.concatenate([g, jnp.zeros_like(g)], axis=1)
    xeo_ref[...] = xe_ref[...] + HSTEP * g


def _eapply(xg, xe, wat, wbt, w2t, sumv, sqv):
    return pl.pallas_call(
        _eapply_body,
        grid=(NEB,),
        in_specs=[pl.BlockSpec((EB, 128), lambda i: (i, 0)),
                  pl.BlockSpec((EB, 64), lambda i: (i, 0)),
                  _wspec((128, 128)), _wspec((64, 128)), _wspec((128, 64)),
                  _wspec((1, 128)), _wspec((1, 128))],
        out_specs=[pl.BlockSpec((EB, 128), lambda i: (i, 0)),
                   pl.BlockSpec((EB, 64), lambda i: (i, 0))],
        out_shape=[jax.ShapeDtypeStruct((E, 128), _f32),
                   jax.ShapeDtypeStruct((E, 64), _f32)],
    )(xg, xe, wat, wbt, w2t, sumv, sqv)


def _pad_rows(w, rows):
    return jnp.concatenate([w, jnp.zeros((rows - w.shape[0], w.shape[1]),
                                         _f32)], axis=0)


def kernel(xn, xe, iInd, jInd, K1Nopen, K2Nopen, K1Eopen, K2Eopen, KNout,
           KE1, KE2, KN1, KN2):
    xn_t = jnp.transpose(xn[0])
    xe_t = jnp.transpose(xe[0])
    ii3 = iInd.astype(jnp.int32).reshape(NW, NCH, CH)
    jj3 = jInd.astype(jnp.int32).reshape(NW, NCH, CH)
    zeros_n = jnp.zeros((NPAD, 128), _f32)

    xn_c = _node_open(xn_t, K1Nopen.T, K2Nopen.T)
    sumv, sqv = _eopen_stats(xe_t, K1Eopen.T)
    xe_c = _eopen_apply(xe_t, K1Eopen.T, K2Eopen.T, sumv, sqv)

    nlayers = KE1.shape[0]
    for i in range(nlayers):
        wat = _pad_rows(jnp.transpose(KE1[i][:, :64] + KE1[i][:, 64:128]),
                        128)
        wbt = jnp.transpose(KE1[i][:, 128:])
        w2t = jnp.transpose(KE2[i])
        xg = _sc_gather(xn_c, ii3)
        sumv, sqv = _estats(xg, xe_c, wat, wbt)
        g, xe_c = _eapply(xg, xe_c, wat, wbt, w2t, sumv, sqv)
        parts = _sc_scatter(g, ii3, jj3, zeros_n)
        xn_c = _node_layer(parts, xn_c,
                           jnp.transpose(KN1[i][:, :64]),
                           jnp.transpose(KN1[i][:, 64:128]),
                           _pad_rows(jnp.transpose(KN1[i][:, 128:]), 128),
                           jnp.transpose(KN2[i]))

    xn_out = _final(xn_c, _pad_rows(KNout.T, 128))
    return jnp.transpose(xn_out)[None], jnp.transpose(xe_c[:, :64])[None]


# speedup vs baseline: 2.6601x; 1.2209x over previous
import jax
import jax.numpy as jnp
from jax import lax
from jax.experimental import pallas as pl
from jax.experimental.pallas import tpu as pltpu
from jax.experimental.pallas import tpu_sc as plsc

N = 10000
E = 160000
EPS = 1e-5
HSTEP = 0.1

NC = 2
NS = 16
NW = NC * NS
EPW = E // NW
CH = 120
NFULL = EPW // CH
TAIL = EPW - NFULL * CH
NCH = NFULL + 1
NPAD = 10240
NPS = NPAD // NS
TRASH = NPAD - 1

EB = 2000
NEB = E // EB

_f32 = jnp.float32


def _mesh():
    return plsc.VectorSubcoreMesh(core_axis_name="c", subcore_axis_name="s",
                                  num_cores=NC, num_subcores=NS)


def _sc_gather_body(table_hbm, idx_hbm, out_hbm, idx_v, buf0, buf1,
                    gs0, gs1, ws0, ws1):
    cid = lax.axis_index("c")
    sid = lax.axis_index("s")
    wid = sid * NC + cid
    base = wid * EPW
    pltpu.sync_copy(idx_hbm.at[wid], idx_v)

    def step(t, carry):
        c0 = 2 * t
        c1 = 2 * t + 1
        h0 = pltpu.async_copy(table_hbm.at[idx_v.at[c0]], buf0, gs0)
        h1 = pltpu.async_copy(table_hbm.at[idx_v.at[c1]], buf1, gs1)
        h0.wait()
        w0 = pltpu.async_copy(buf0, out_hbm.at[pl.ds(base + c0 * CH, CH)],
                              ws0)
        h1.wait()
        w1 = pltpu.async_copy(buf1, out_hbm.at[pl.ds(base + c1 * CH, CH)],
                              ws1)
        w0.wait()
        w1.wait()
        return carry

    lax.fori_loop(0, NFULL // 2, step, 0)
    c0 = NFULL - 1
    h0 = pltpu.async_copy(table_hbm.at[idx_v.at[c0]], buf0, gs0)
    h1 = pltpu.async_copy(table_hbm.at[idx_v.at[NFULL]], buf1, gs1)
    h0.wait()
    w0 = pltpu.async_copy(buf0, out_hbm.at[pl.ds(base + c0 * CH, CH)], ws0)
    h1.wait()
    w1 = pltpu.async_copy(buf1.at[pl.ds(0, TAIL)],
                          out_hbm.at[pl.ds(base + NFULL * CH, TAIL)], ws1)
    w0.wait()
    w1.wait()


def _sc_gather(table, idx3):
    return pl.kernel(
        _sc_gather_body,
        out_type=jax.ShapeDtypeStruct((E, 128), _f32),
        mesh=_mesh(),
        scratch_types=[
            pltpu.VMEM((NCH, CH), jnp.int32),
            pltpu.VMEM((CH, 128), _f32),
            pltpu.VMEM((CH, 128), _f32),
            pltpu.SemaphoreType.DMA,
            pltpu.SemaphoreType.DMA,
            pltpu.SemaphoreType.DMA,
            pltpu.SemaphoreType.DMA,
        ],
    )(table, idx3)


def _scatter_phase(g_hbm, idx_v, acc, buf0, buf1, rs0, rs1, ss0, ss1, base):
    def step(t, carry):
        c0 = 2 * t
        c1 = 2 * t + 1
        r0 = pltpu.async_copy(g_hbm.at[pl.ds(base + c0 * CH, CH)], buf0, rs0)
        r1 = pltpu.async_copy(g_hbm.at[pl.ds(base + c1 * CH, CH)], buf1, rs1)
        r0.wait()
        s0 = pltpu.async_copy(buf0, acc.at[idx_v.at[c0]], ss0, add=True)
        r1.wait()
        s1 = pltpu.async_copy(buf1, acc.at[idx_v.at[c1]], ss1, add=True)
        s0.wait()
        s1.wait()
        return carry

    lax.fori_loop(0, NFULL // 2, step, 0)
    c0 = NFULL - 1
    r0 = pltpu.async_copy(g_hbm.at[pl.ds(base + c0 * CH, CH)], buf0, rs0)
    r1 = pltpu.async_copy(g_hbm.at[pl.ds(base + NFULL * CH, TAIL)],
                          buf1.at[pl.ds(0, TAIL)], rs1)
    r0.wait()
    s0 = pltpu.async_copy(buf0, acc.at[idx_v.at[c0]], ss0, add=True)
    r1.wait()
    s1 = pltpu.async_copy(buf1, acc.at[idx_v.at[NFULL]], ss1, add=True)
    s0.wait()
    s1.wait()


def _sc_scatter_body(g_hbm, ii_hbm, jj_hbm, z_hbm, out_hbm,
                     ii_v, jj_v, buf0, buf1, acc,
                     rs0, rs1, ss0, ss1):
    cid = lax.axis_index("c")
    sid = lax.axis_index("s")
    wid = sid * NC + cid
    base = wid * EPW
    sl = pl.ds(sid * NPS, NPS)
    pltpu.sync_copy(ii_hbm.at[wid], ii_v)
    pltpu.sync_copy(jj_hbm.at[wid], jj_v)
    pltpu.sync_copy(z_hbm, buf0)
    for k in range(NPS // CH):
        pltpu.sync_copy(buf0, acc.at[pl.ds(sid * NPS + k * CH, CH)])
    rem = NPS - (NPS // CH) * CH
    if rem:
        pltpu.sync_copy(buf0.at[pl.ds(0, rem)],
                        acc.at[pl.ds(sid * NPS + (NPS // CH) * CH, rem)])
    plsc.subcore_barrier()
    _scatter_phase(g_hbm, ii_v, acc, buf0, buf1, rs0, rs1, ss0, ss1, base)
    plsc.subcore_barrier()
    pltpu.sync_copy(acc.at[sl], out_hbm.at[cid, 0, sl])
    plsc.subcore_barrier()
    _scatter_phase(g_hbm, jj_v, acc, buf0, buf1, rs0, rs1, ss0, ss1, base)
    plsc.subcore_barrier()
    pltpu.sync_copy(acc.at[sl], out_hbm.at[cid, 1, sl])


def _sc_scatter(g, ii3, jj3, zeros_n):
    return pl.kernel(
        _sc_scatter_body,
        out_type=jax.ShapeDtypeStruct((NC, 2, NPAD, 128), _f32),
        mesh=_mesh(),
        scratch_types=[
            pltpu.VMEM((NCH, CH), jnp.int32),
            pltpu.VMEM((NCH, CH), jnp.int32),
            pltpu.VMEM((CH, 128), _f32),
            pltpu.VMEM((CH, 128), _f32),
            pltpu.VMEM_SHARED((NPAD, 128), _f32),
            pltpu.SemaphoreType.DMA,
            pltpu.SemaphoreType.DMA,
            pltpu.SemaphoreType.DMA,
            pltpu.SemaphoreType.DMA,
        ],
    )(g, ii3, jj3, zeros_n)


def _ln_relu(h):
    cnt = float(h.shape[0] * h.shape[1])
    m = jnp.sum(h) / cnt
    v = jnp.sum((h - m) * (h - m)) / cnt
    return jnp.maximum((h - m) * lax.rsqrt(v + EPS), 0.0)


def _node_open_body(xn_ref, w1t_ref, w2t_ref, out_ref):
    h = jnp.dot(xn_ref[...], w1t_ref[...], preferred_element_type=_f32)
    h = _ln_relu(h)
    r = jnp.dot(h, w2t_ref[...], preferred_element_type=_f32)
    out_ref[...] = jnp.concatenate([r, jnp.zeros_like(r)], axis=1)


def _node_open(xn_t, w1t, w2t):
    return pl.pallas_call(
        _node_open_body,
        out_shape=jax.ShapeDtypeStruct((N, 128), _f32),
    )(xn_t, w1t, w2t)


def _node_layer_body(parts_ref, xn_ref, wat_ref, wbt_ref, wct_ref, w2t_ref,
                     out_ref):
    p = parts_ref[...]
    s_i = p[0, 0, :N, :64] + p[1, 0, :N, :64]
    s_j = (p[0, 1, :N, :64] + p[1, 1, :N, :64]) - s_i
    ave = 0.5 * (s_i + s_j)
    div = s_i - s_j
    xn = xn_ref[...]
    h = jnp.dot(ave, wat_ref[...], preferred_element_type=_f32)
    h = h + jnp.dot(div, wbt_ref[...], preferred_element_type=_f32)
    h = h + jnp.dot(xn, wct_ref[...], preferred_element_type=_f32)
    h = _ln_relu(h)
    r = jnp.dot(h, w2t_ref[...], preferred_element_type=_f32)
    out_ref[...] = xn + HSTEP * jnp.concatenate(
        [r, jnp.zeros_like(r)], axis=1)


def _node_layer(parts, xn, wat, wbt, wct, w2t):
    return pl.pallas_call(
        _node_layer_body,
        out_shape=jax.ShapeDtypeStruct((N, 128), _f32),
    )(parts, xn, wat, wbt, wct, w2t)


def _final_body(xn_ref, wt_ref, out_ref):
    out_ref[...] = jnp.dot(xn_ref[...], wt_ref[...],
                           preferred_element_type=_f32)


def _final(xn, wt):
    return pl.pallas_call(
        _final_body,
        out_shape=jax.ShapeDtypeStruct((N, 128), _f32),
    )(xn, wt)


def _wspec(shape):
    return pl.BlockSpec(shape, lambda i: (0, 0))


def _eopen_stats_body(xe_ref, w1t_ref, sum_ref, sq_ref):
    i = pl.program_id(0)
    h = jnp.dot(xe_ref[...], w1t_ref[...], preferred_element_type=_f32)

    @pl.when(i == 0)
    def _():
        sum_ref[...] = jnp.zeros_like(sum_ref)
        sq_ref[...] = jnp.zeros_like(sq_ref)

    sum_ref[...] += jnp.sum(h, axis=0, keepdims=True)
    sq_ref[...] += jnp.sum(h * h, axis=0, keepdims=True)


def _eopen_stats(xe_t, w1t):
    return pl.pallas_call(
        _eopen_stats_body,
        grid=(NEB,),
        in_specs=[pl.BlockSpec((EB, 16), lambda i: (i, 0)), _wspec((16, 64))],
        out_specs=[pl.BlockSpec((1, 64), lambda i: (0, 0)),
                   pl.BlockSpec((1, 64), lambda i: (0, 0))],
        out_shape=[jax.ShapeDtypeStruct((1, 64), _f32),
                   jax.ShapeDtypeStruct((1, 64), _f32)],
    )(xe_t, w1t)


def _eopen_apply_body(xe_ref, w1t_ref, w2t_ref, sum_ref, sq_ref, out_ref):
    cnt = float(E * 64)
    m = jnp.sum(sum_ref[...]) / cnt
    v = jnp.sum(sq_ref[...]) / cnt - m * m
    h = jnp.dot(xe_ref[...], w1t_ref[...], preferred_element_type=_f32)
    h = jnp.maximum((h - m) * lax.rsqrt(v + EPS), 0.0)
    out_ref[...] = jnp.dot(h, w2t_ref[...], preferred_element_type=_f32)


def _eopen_apply(xe_t, w1t, w2t, sumv, sqv):
    return pl.pallas_call(
        _eopen_apply_body,
        grid=(NEB,),
        in_specs=[pl.BlockSpec((EB, 16), lambda i: (i, 0)),
                  _wspec((16, 64)), _wspec((64, 64)),
                  _wspec((1, 64)), _wspec((1, 64))],
        out_specs=pl.BlockSpec((EB, 64), lambda i: (i, 0)),
        out_shape=jax.ShapeDtypeStruct((E, 64), _f32),
    )(xe_t, w1t, w2t, sumv, sqv)


def _estats_body(xg_ref, xe_ref, wat_ref, wbt_ref, sum_ref, sq_ref):
    i = pl.program_id(0)
    h = jnp.dot(xg_ref[...], wat_ref[...], preferred_element_type=_f32)
    h = h + jnp.dot(xe_ref[...], wbt_ref[...], preferred_element_type=_f32)

    @pl.when(i == 0)
    def _():
        sum_ref[...] = jnp.zeros_like(sum_ref)
        sq_ref[...] = jnp.zeros_like(sq_ref)

    sum_ref[...] += jnp.sum(h, axis=0, keepdims=True)
    sq_ref[...] += jnp.sum(h * h, axis=0, keepdims=True)


def _estats(xg, xe, wat, wbt):
    return pl.pallas_call(
        _estats_body,
        grid=(NEB,),
        in_specs=[pl.BlockSpec((EB, 128), lambda i: (i, 0)),
                  pl.BlockSpec((EB, 64), lambda i: (i, 0)),
                  _wspec((128, 128)), _wspec((64, 128))],
        out_specs=[pl.BlockSpec((1, 128), lambda i: (0, 0)),
                   pl.BlockSpec((1, 128), lambda i: (0, 0))],
        out_shape=[jax.ShapeDtypeStruct((1, 128), _f32),
                   jax.ShapeDtypeStruct((1, 128), _f32)],
    )(xg, xe, wat, wbt)


def _eapply_body(xg_ref, xe_ref, wat_ref, wbt_ref, w2t_ref, sum_ref, sq_ref,
                 g_ref, xeo_ref):
    cnt = float(E * 128)
    m = jnp.sum(sum_ref[...]) / cnt
    v = jnp.sum(sq_ref[...]) / cnt - m * m
    h = jnp.dot(xg_ref[...], wat_ref[...], preferred_element_type=_f32)
    h = h + jnp.dot(xe_ref[...], wbt_ref[...], preferred_element_type=_f32)
    h = jnp.maximum((h - m) * lax.rsqrt(v + EPS), 0.0)
    g = jnp.dot(h, w2t_ref[...], preferred_element_type=_f32)
    g_ref[...] = jnp.concatenate([g, jnp.zeros_like(g)], axis=1)
    xeo_ref[...] = xe_ref[...] + HSTEP * g


def _eapply(xg, xe, wat, wbt, w2t, sumv, sqv):
    return pl.pallas_call(
        _eapply_body,
        grid=(NEB,),
        in_specs=[pl.BlockSpec((EB, 128), lambda i: (i, 0)),
                  pl.BlockSpec((EB, 64), lambda i: (i, 0)),
                  _wspec((128, 128)), _wspec((64, 128)), _wspec((128, 64)),
                  _wspec((1, 128)), _wspec((1, 128))],
        out_specs=[pl.BlockSpec((EB, 128), lambda i: (i, 0)),
                   pl.BlockSpec((EB, 64), lambda i: (i, 0))],
        out_shape=[jax.ShapeDtypeStruct((E, 128), _f32),
                   jax.ShapeDtypeStruct((E, 64), _f32)],
    )(xg, xe, wat, wbt, w2t, sumv, sqv)


def _pad_rows(w, rows):
    return jnp.concatenate([w, jnp.zeros((rows - w.shape[0], w.shape[1]),
                                         _f32)], axis=0)


def kernel(xn, xe, iInd, jInd, K1Nopen, K2Nopen, K1Eopen, K2Eopen, KNout,
           KE1, KE2, KN1, KN2):
    xn_t = jnp.transpose(xn[0])
    xe_t = jnp.transpose(xe[0])

    def _pad_idx(ind, fill):
        w = ind.astype(jnp.int32).reshape(NW, EPW)
        pad = jnp.full((NW, NCH * CH - EPW), fill, jnp.int32)
        return jnp.concatenate([w, pad], axis=1).reshape(NW, NCH, CH)

    gi3 = _pad_idx(iInd, 0)
    ii3 = _pad_idx(iInd, TRASH)
    jj3 = _pad_idx(jInd, TRASH)
    zeros_n = jnp.zeros((CH, 128), _f32)

    xn_c = _node_open(xn_t, K1Nopen.T, K2Nopen.T)
    sumv, sqv = _eopen_stats(xe_t, K1Eopen.T)
    xe_c = _eopen_apply(xe_t, K1Eopen.T, K2Eopen.T, sumv, sqv)

    nlayers = KE1.shape[0]
    for i in range(nlayers):
        wat = _pad_rows(jnp.transpose(KE1[i][:, :64] + KE1[i][:, 64:128]),
                        128)
        wbt = jnp.transpose(KE1[i][:, 128:])
        w2t = jnp.transpose(KE2[i])
        xg = _sc_gather(xn_c, gi3)
        sumv, sqv = _estats(xg, xe_c, wat, wbt)
        g, xe_c = _eapply(xg, xe_c, wat, wbt, w2t, sumv, sqv)
        parts = _sc_scatter(g, ii3, jj3, zeros_n)
        xn_c = _node_layer(parts, xn_c,
                           jnp.transpose(KN1[i][:, :64]),
                           jnp.transpose(KN1[i][:, 64:128]),
                           _pad_rows(jnp.transpose(KN1[i][:, 128:]), 128),
                           jnp.transpose(KN2[i]))

    xn_out = _final(xn_c, _pad_rows(KNout.T, 128))
    return jnp.transpose(xn_out)[None], jnp.transpose(xe_c[:, :64])[None]


# speedup vs baseline: 2.6634x; 1.0013x over previous
import jax
import jax.numpy as jnp
from jax import lax
from jax.experimental import pallas as pl
from jax.experimental.pallas import tpu as pltpu
from jax.experimental.pallas import tpu_sc as plsc

N = 10000
E = 160000
EPS = 1e-5
HSTEP = 0.1

NC = 2
NS = 16
NW = NC * NS
EPW = E // NW
CH = 120
NFULL = EPW // CH
TAIL = EPW - NFULL * CH
NCH = NFULL + 1
NPAD = 10240
NPS = NPAD // NS
TRASH = NPAD - 1

EB = 2000
NEB = E // EB

_f32 = jnp.float32


def _mesh():
    return plsc.VectorSubcoreMesh(core_axis_name="c", subcore_axis_name="s",
                                  num_cores=NC, num_subcores=NS)


def _sc_gather_body(table_hbm, idx_hbm, out_hbm, idx_v, buf0, buf1,
                    gs0, gs1, ws0, ws1):
    cid = lax.axis_index("c")
    sid = lax.axis_index("s")
    wid = sid * NC + cid
    base = wid * EPW
    pltpu.sync_copy(idx_hbm.at[wid], idx_v)

    def step(t, carry):
        c0 = 2 * t
        c1 = 2 * t + 1
        h0 = pltpu.async_copy(table_hbm.at[idx_v.at[c0]], buf0, gs0)
        h1 = pltpu.async_copy(table_hbm.at[idx_v.at[c1]], buf1, gs1)
        h0.wait()
        w0 = pltpu.async_copy(buf0, out_hbm.at[pl.ds(base + c0 * CH, CH)],
                              ws0)
        h1.wait()
        w1 = pltpu.async_copy(buf1, out_hbm.at[pl.ds(base + c1 * CH, CH)],
                              ws1)
        w0.wait()
        w1.wait()
        return carry

    lax.fori_loop(0, NFULL // 2, step, 0)
    c0 = NFULL - 1
    h0 = pltpu.async_copy(table_hbm.at[idx_v.at[c0]], buf0, gs0)
    h1 = pltpu.async_copy(table_hbm.at[idx_v.at[NFULL]], buf1, gs1)
    h0.wait()
    w0 = pltpu.async_copy(buf0, out_hbm.at[pl.ds(base + c0 * CH, CH)], ws0)
    h1.wait()
    w1 = pltpu.async_copy(buf1.at[pl.ds(0, TAIL)],
                          out_hbm.at[pl.ds(base + NFULL * CH, TAIL)], ws1)
    w0.wait()
    w1.wait()


def _sc_gather(table, idx3):
    return pl.kernel(
        _sc_gather_body,
        out_type=jax.ShapeDtypeStruct((E, 128), _f32),
        mesh=_mesh(),
        scratch_types=[
            pltpu.VMEM((NCH, CH), jnp.int32),
            pltpu.VMEM((CH, 128), _f32),
            pltpu.VMEM((CH, 128), _f32),
            pltpu.SemaphoreType.DMA,
            pltpu.SemaphoreType.DMA,
            pltpu.SemaphoreType.DMA,
            pltpu.SemaphoreType.DMA,
        ],
    )(table, idx3)


def _scatter_phase(g_hbm, idx_v, acc, buf0, buf1, rs0, rs1, ss0, ss1, base):
    def step(t, carry):
        c0 = 2 * t
        c1 = 2 * t + 1
        r0 = pltpu.async_copy(g_hbm.at[pl.ds(base + c0 * CH, CH)], buf0, rs0)
        r1 = pltpu.async_copy(g_hbm.at[pl.ds(base + c1 * CH, CH)], buf1, rs1)
        r0.wait()
        s0 = pltpu.async_copy(buf0, acc.at[idx_v.at[c0]], ss0, add=True)
        r1.wait()
        s1 = pltpu.async_copy(buf1, acc.at[idx_v.at[c1]], ss1, add=True)
        s0.wait()
        s1.wait()
        return carry

    lax.fori_loop(0, NFULL // 2, step, 0)
    c0 = NFULL - 1
    r0 = pltpu.async_copy(g_hbm.at[pl.ds(base + c0 * CH, CH)], buf0, rs0)
    r1 = pltpu.async_copy(g_hbm.at[pl.ds(base + NFULL * CH, TAIL)],
                          buf1.at[pl.ds(0, TAIL)], rs1)
    r0.wait()
    s0 = pltpu.async_copy(buf0, acc.at[idx_v.at[c0]], ss0, add=True)
    r1.wait()
    s1 = pltpu.async_copy(buf1, acc.at[idx_v.at[NFULL]], ss1, add=True)
    s0.wait()
    s1.wait()


def _sc_scatter_body(g_hbm, ii_hbm, jj_hbm, z_hbm, out_hbm,
                     ii_v, jj_v, buf0, buf1, acc,
                     rs0, rs1, ss0, ss1):
    cid = lax.axis_index("c")
    sid = lax.axis_index("s")
    wid = sid * NC + cid
    base = wid * EPW
    sl = pl.ds(sid * NPS, NPS)
    pltpu.sync_copy(ii_hbm.at[wid], ii_v)
    pltpu.sync_copy(jj_hbm.at[wid], jj_v)
    pltpu.sync_copy(z_hbm, buf0)
    for k in range(NPS // CH):
        pltpu.sync_copy(buf0, acc.at[pl.ds(sid * NPS + k * CH, CH)])
    rem = NPS - (NPS // CH) * CH
    if rem:
        pltpu.sync_copy(buf0.at[pl.ds(0, rem)],
                        acc.at[pl.ds(sid * NPS + (NPS // CH) * CH, rem)])
    plsc.subcore_barrier()
    _scatter_phase(g_hbm, ii_v, acc, buf0, buf1, rs0, rs1, ss0, ss1, base)
    plsc.subcore_barrier()
    pltpu.sync_copy(acc.at[sl], out_hbm.at[cid, 0, sl])
    plsc.subcore_barrier()
    _scatter_phase(g_hbm, jj_v, acc, buf0, buf1, rs0, rs1, ss0, ss1, base)
    plsc.subcore_barrier()
    pltpu.sync_copy(acc.at[sl], out_hbm.at[cid, 1, sl])


def _sc_scatter(g, ii3, jj3, zeros_n):
    return pl.kernel(
        _sc_scatter_body,
        out_type=jax.ShapeDtypeStruct((NC, 2, NPAD, 128), _f32),
        mesh=_mesh(),
        scratch_types=[
            pltpu.VMEM((NCH, CH), jnp.int32),
            pltpu.VMEM((NCH, CH), jnp.int32),
            pltpu.VMEM((CH, 128), _f32),
            pltpu.VMEM((CH, 128), _f32),
            pltpu.VMEM_SHARED((NPAD, 128), _f32),
            pltpu.SemaphoreType.DMA,
            pltpu.SemaphoreType.DMA,
            pltpu.SemaphoreType.DMA,
            pltpu.SemaphoreType.DMA,
        ],
    )(g, ii3, jj3, zeros_n)


def _ln_relu(h):
    cnt = float(h.shape[0] * h.shape[1])
    m = jnp.sum(h) / cnt
    v = jnp.sum((h - m) * (h - m)) / cnt
    return jnp.maximum((h - m) * lax.rsqrt(v + EPS), 0.0)


def _node_open_body(xn_ref, w1t_ref, w2t_ref, out_ref):
    h = jnp.dot(xn_ref[...], w1t_ref[...], preferred_element_type=_f32)
    h = _ln_relu(h)
    r = jnp.dot(h, w2t_ref[...], preferred_element_type=_f32)
    out_ref[...] = jnp.concatenate([r, jnp.zeros_like(r)], axis=1)


def _node_open(xn_t, w1t, w2t):
    return pl.pallas_call(
        _node_open_body,
        out_shape=jax.ShapeDtypeStruct((N, 128), _f32),
    )(xn_t, w1t, w2t)


def _node_layer_body(parts_ref, xn_ref, wat_ref, wbt_ref, wct_ref, w2t_ref,
                     out_ref):
    p = parts_ref[...]
    s_i = p[0, 0, :N, :64] + p[1, 0, :N, :64]
    s_j = (p[0, 1, :N, :64] + p[1, 1, :N, :64]) - s_i
    ave = 0.5 * (s_i + s_j)
    div = s_i - s_j
    xn = xn_ref[...]
    h = jnp.dot(ave, wat_ref[...], preferred_element_type=_f32)
    h = h + jnp.dot(div, wbt_ref[...], preferred_element_type=_f32)
    h = h + jnp.dot(xn, wct_ref[...], preferred_element_type=_f32)
    h = _ln_relu(h)
    r = jnp.dot(h, w2t_ref[...], preferred_element_type=_f32)
    out_ref[...] = xn + HSTEP * jnp.concatenate(
        [r, jnp.zeros_like(r)], axis=1)


def _node_layer(parts, xn, wat, wbt, wct, w2t):
    return pl.pallas_call(
        _node_layer_body,
        out_shape=jax.ShapeDtypeStruct((N, 128), _f32),
    )(parts, xn, wat, wbt, wct, w2t)


def _final_body(xn_ref, wt_ref, out_ref):
    out_ref[...] = jnp.dot(xn_ref[...], wt_ref[...],
                           preferred_element_type=_f32)


def _final(xn, wt):
    return pl.pallas_call(
        _final_body,
        out_shape=jax.ShapeDtypeStruct((N, 128), _f32),
    )(xn, wt)


def _wspec(shape):
    return pl.BlockSpec(shape, lambda i: (0, 0))


def _eopen_stats_body(xe_ref, w1t_ref, sum_ref, sq_ref):
    i = pl.program_id(0)
    h = jnp.dot(xe_ref[...], w1t_ref[...], preferred_element_type=_f32)

    @pl.when(i == 0)
    def _():
        sum_ref[...] = jnp.zeros_like(sum_ref)
        sq_ref[...] = jnp.zeros_like(sq_ref)

    sum_ref[...] += jnp.sum(h, axis=0, keepdims=True)
    sq_ref[...] += jnp.sum(h * h, axis=0, keepdims=True)


def _eopen_stats(xe_t, w1t):
    return pl.pallas_call(
        _eopen_stats_body,
        grid=(NEB,),
        in_specs=[pl.BlockSpec((EB, 16), lambda i: (i, 0)), _wspec((16, 64))],
        out_specs=[pl.BlockSpec((1, 64), lambda i: (0, 0)),
                   pl.BlockSpec((1, 64), lambda i: (0, 0))],
        out_shape=[jax.ShapeDtypeStruct((1, 64), _f32),
                   jax.ShapeDtypeStruct((1, 64), _f32)],
    )(xe_t, w1t)


def _eopen_apply_body(xe_ref, w1t_ref, w2t_ref, sum_ref, sq_ref, out_ref):
    cnt = float(E * 64)
    m = jnp.sum(sum_ref[...]) / cnt
    v = jnp.sum(sq_ref[...]) / cnt - m * m
    h = jnp.dot(xe_ref[...], w1t_ref[...], preferred_element_type=_f32)
    h = jnp.maximum((h - m) * lax.rsqrt(v + EPS), 0.0)
    out_ref[...] = jnp.dot(h, w2t_ref[...], preferred_element_type=_f32)


def _eopen_apply(xe_t, w1t, w2t, sumv, sqv):
    return pl.pallas_call(
        _eopen_apply_body,
        grid=(NEB,),
        in_specs=[pl.BlockSpec((EB, 16), lambda i: (i, 0)),
                  _wspec((16, 64)), _wspec((64, 64)),
                  _wspec((1, 64)), _wspec((1, 64))],
        out_specs=pl.BlockSpec((EB, 64), lambda i: (i, 0)),
        out_shape=jax.ShapeDtypeStruct((E, 64), _f32),
    )(xe_t, w1t, w2t, sumv, sqv)


def _estats_body(xg_ref, xe_ref, wat_ref, wbt_ref, sum_ref, sq_ref):
    i = pl.program_id(0)
    h = jnp.dot(xg_ref[:, :64], wat_ref[...], preferred_element_type=_f32)
    h = h + jnp.dot(xe_ref[...], wbt_ref[...], preferred_element_type=_f32)

    @pl.when(i == 0)
    def _():
        sum_ref[...] = jnp.zeros_like(sum_ref)
        sq_ref[...] = jnp.zeros_like(sq_ref)

    sum_ref[...] += jnp.sum(h, axis=0, keepdims=True)
    sq_ref[...] += jnp.sum(h * h, axis=0, keepdims=True)


def _estats(xg, xe, wat, wbt):
    return pl.pallas_call(
        _estats_body,
        grid=(NEB,),
        in_specs=[pl.BlockSpec((EB, 128), lambda i: (i, 0)),
                  pl.BlockSpec((EB, 64), lambda i: (i, 0)),
                  _wspec((64, 128)), _wspec((64, 128))],
        out_specs=[pl.BlockSpec((1, 128), lambda i: (0, 0)),
                   pl.BlockSpec((1, 128), lambda i: (0, 0))],
        out_shape=[jax.ShapeDtypeStruct((1, 128), _f32),
                   jax.ShapeDtypeStruct((1, 128), _f32)],
    )(xg, xe, wat, wbt)


def _eapply_body(xg_ref, xe_ref, wat_ref, wbt_ref, w2t_ref, sum_ref, sq_ref,
                 g_ref, xeo_ref):
    cnt = float(E * 128)
    m = jnp.sum(sum_ref[...]) / cnt
    v = jnp.sum(sq_ref[...]) / cnt - m * m
    h = jnp.dot(xg_ref[:, :64], wat_ref[...], preferred_element_type=_f32)
    h = h + jnp.dot(xe_ref[...], wbt_ref[...], preferred_element_type=_f32)
    h = jnp.maximum((h - m) * lax.rsqrt(v + EPS), 0.0)
    g = jnp.dot(h, w2t_ref[...], preferred_element_type=_f32)
    g_ref[...] = jnp.concatenate([g, jnp.zeros_like(g)], axis=1)
    xeo_ref[...] = xe_ref[...] + HSTEP * g


def _eapply(xg, xe, wat, wbt, w2t, sumv, sqv):
    return pl.pallas_call(
        _eapply_body,
        grid=(NEB,),
        in_specs=[pl.BlockSpec((EB, 128), lambda i: (i, 0)),
                  pl.BlockSpec((EB, 64), lambda i: (i, 0)),
                  _wspec((64, 128)), _wspec((64, 128)), _wspec((128, 64)),
                  _wspec((1, 128)), _wspec((1, 128))],
        out_specs=[pl.BlockSpec((EB, 128), lambda i: (i, 0)),
                   pl.BlockSpec((EB, 64), lambda i: (i, 0))],
        out_shape=[jax.ShapeDtypeStruct((E, 128), _f32),
                   jax.ShapeDtypeStruct((E, 64), _f32)],
    )(xg, xe, wat, wbt, w2t, sumv, sqv)


def _pad_rows(w, rows):
    return jnp.concatenate([w, jnp.zeros((rows - w.shape[0], w.shape[1]),
                                         _f32)], axis=0)


def kernel(xn, xe, iInd, jInd, K1Nopen, K2Nopen, K1Eopen, K2Eopen, KNout,
           KE1, KE2, KN1, KN2):
    xn_t = jnp.transpose(xn[0])
    xe_t = jnp.transpose(xe[0])

    def _pad_idx(ind, fill):
        w = ind.astype(jnp.int32).reshape(NW, EPW)
        pad = jnp.full((NW, NCH * CH - EPW), fill, jnp.int32)
        return jnp.concatenate([w, pad], axis=1).reshape(NW, NCH, CH)

    gi3 = _pad_idx(iInd, 0)
    ii3 = _pad_idx(iInd, TRASH)
    jj3 = _pad_idx(jInd, TRASH)
    zeros_n = jnp.zeros((CH, 128), _f32)

    xn_c = _node_open(xn_t, K1Nopen.T, K2Nopen.T)
    xg = _sc_gather(xn_c, gi3)
    sumv, sqv = _eopen_stats(xe_t, K1Eopen.T)
    xe_c = _eopen_apply(xe_t, K1Eopen.T, K2Eopen.T, sumv, sqv)

    nlayers = KE1.shape[0]
    for i in range(nlayers):
        wat = jnp.transpose(KE1[i][:, :64] + KE1[i][:, 64:128])
        wbt = jnp.transpose(KE1[i][:, 128:])
        w2t = jnp.transpose(KE2[i])
        sumv, sqv = _estats(xg, xe_c, wat, wbt)
        g, xe_c = _eapply(xg, xe_c, wat, wbt, w2t, sumv, sqv)
        parts = _sc_scatter(g, ii3, jj3, zeros_n)
        xn_c = _node_layer(parts, xn_c,
                           jnp.transpose(KN1[i][:, :64]),
                           jnp.transpose(KN1[i][:, 64:128]),
                           _pad_rows(jnp.transpose(KN1[i][:, 128:]), 128),
                           jnp.transpose(KN2[i]))
        if i + 1 < nlayers:
            xg = _sc_gather(xn_c, gi3)

    xn_out = _final(xn_c, _pad_rows(KNout.T, 128))
    return jnp.transpose(xn_out)[None], jnp.transpose(xe_c[:, :64])[None]


# speedup vs baseline: 2.6659x; 1.0009x over previous
import jax
import jax.numpy as jnp
from jax import lax
from jax.experimental import pallas as pl
from jax.experimental.pallas import tpu as pltpu
from jax.experimental.pallas import tpu_sc as plsc

N = 10000
E = 160000
EPS = 1e-5
HSTEP = 0.1

NC = 2
NS = 16
NW = NC * NS
EPW = E // NW
CH = 120
NFULL = EPW // CH
TAIL = EPW - NFULL * CH
NCH = NFULL + 1
NPAD = 10240
NPS = NPAD // NS
TRASH = NPAD - 1

EB = 2000
NEB = E // EB

_f32 = jnp.float32


def _mesh():
    return plsc.VectorSubcoreMesh(core_axis_name="c", subcore_axis_name="s",
                                  num_cores=NC, num_subcores=NS)


def _sc_gather_body(table_hbm, idx_hbm, out_hbm, idx_v,
                    buf0, buf1, buf2, buf3,
                    gs0, gs1, gs2, gs3, ws0, ws1, ws2, ws3):
    cid = lax.axis_index("c")
    sid = lax.axis_index("s")
    wid = sid * NC + cid
    base = wid * EPW
    pltpu.sync_copy(idx_hbm.at[wid], idx_v)
    bufs = (buf0, buf1, buf2, buf3)
    gss = (gs0, gs1, gs2, gs3)
    wss = (ws0, ws1, ws2, ws3)

    def step(t, carry):
        hs = [pltpu.async_copy(table_hbm.at[idx_v.at[4 * t + k]],
                               bufs[k], gss[k]) for k in range(4)]
        ws = []
        for k in range(4):
            hs[k].wait()
            ws.append(pltpu.async_copy(
                bufs[k], out_hbm.at[pl.ds(base + (4 * t + k) * CH, CH)],
                wss[k]))
        for w in ws:
            w.wait()
        return carry

    lax.fori_loop(0, NFULL // 4, step, 0)
    rem = NFULL - (NFULL // 4) * 4
    hs = [pltpu.async_copy(table_hbm.at[idx_v.at[NFULL - rem + k]],
                           bufs[k], gss[k]) for k in range(rem)]
    ht = pltpu.async_copy(table_hbm.at[idx_v.at[NFULL]], bufs[rem], gss[rem])
    ws = []
    for k in range(rem):
        hs[k].wait()
        ws.append(pltpu.async_copy(
            bufs[k], out_hbm.at[pl.ds(base + (NFULL - rem + k) * CH, CH)],
            wss[k]))
    ht.wait()
    ws.append(pltpu.async_copy(bufs[rem].at[pl.ds(0, TAIL)],
                               out_hbm.at[pl.ds(base + NFULL * CH, TAIL)],
                               wss[rem]))
    for w in ws:
        w.wait()


def _sc_gather(table, idx3):
    return pl.kernel(
        _sc_gather_body,
        out_type=jax.ShapeDtypeStruct((E, 128), _f32),
        mesh=_mesh(),
        scratch_types=[
            pltpu.VMEM((NCH, CH), jnp.int32),
            pltpu.VMEM((CH, 128), _f32),
            pltpu.VMEM((CH, 128), _f32),
            pltpu.VMEM((CH, 128), _f32),
            pltpu.VMEM((CH, 128), _f32),
            pltpu.SemaphoreType.DMA,
            pltpu.SemaphoreType.DMA,
            pltpu.SemaphoreType.DMA,
            pltpu.SemaphoreType.DMA,
            pltpu.SemaphoreType.DMA,
            pltpu.SemaphoreType.DMA,
            pltpu.SemaphoreType.DMA,
            pltpu.SemaphoreType.DMA,
        ],
    )(table, idx3)


def _scatter_phase(g_hbm, idx_v, acc, buf0, buf1, rs0, rs1, ss0, ss1, base):
    def step(t, carry):
        c0 = 2 * t
        c1 = 2 * t + 1
        r0 = pltpu.async_copy(g_hbm.at[pl.ds(base + c0 * CH, CH)], buf0, rs0)
        r1 = pltpu.async_copy(g_hbm.at[pl.ds(base + c1 * CH, CH)], buf1, rs1)
        r0.wait()
        s0 = pltpu.async_copy(buf0, acc.at[idx_v.at[c0]], ss0, add=True)
        r1.wait()
        s1 = pltpu.async_copy(buf1, acc.at[idx_v.at[c1]], ss1, add=True)
        s0.wait()
        s1.wait()
        return carry

    lax.fori_loop(0, NFULL // 2, step, 0)
    c0 = NFULL - 1
    r0 = pltpu.async_copy(g_hbm.at[pl.ds(base + c0 * CH, CH)], buf0, rs0)
    r1 = pltpu.async_copy(g_hbm.at[pl.ds(base + NFULL * CH, TAIL)],
                          buf1.at[pl.ds(0, TAIL)], rs1)
    r0.wait()
    s0 = pltpu.async_copy(buf0, acc.at[idx_v.at[c0]], ss0, add=True)
    r1.wait()
    s1 = pltpu.async_copy(buf1, acc.at[idx_v.at[NFULL]], ss1, add=True)
    s0.wait()
    s1.wait()


def _sc_scatter_body(g_hbm, ii_hbm, jj_hbm, z_hbm, out_hbm,
                     ii_v, jj_v, buf0, buf1, acc,
                     rs0, rs1, ss0, ss1):
    cid = lax.axis_index("c")
    sid = lax.axis_index("s")
    wid = sid * NC + cid
    base = wid * EPW
    sl = pl.ds(sid * NPS, NPS)
    pltpu.sync_copy(ii_hbm.at[wid], ii_v)
    pltpu.sync_copy(jj_hbm.at[wid], jj_v)
    pltpu.sync_copy(z_hbm, buf0)
    for k in range(NPS // CH):
        pltpu.sync_copy(buf0, acc.at[pl.ds(sid * NPS + k * CH, CH)])
    rem = NPS - (NPS // CH) * CH
    if rem:
        pltpu.sync_copy(buf0.at[pl.ds(0, rem)],
                        acc.at[pl.ds(sid * NPS + (NPS // CH) * CH, rem)])
    plsc.subcore_barrier()
    _scatter_phase(g_hbm, ii_v, acc, buf0, buf1, rs0, rs1, ss0, ss1, base)
    plsc.subcore_barrier()
    pltpu.sync_copy(acc.at[sl], out_hbm.at[cid, 0, sl])
    plsc.subcore_barrier()
    _scatter_phase(g_hbm, jj_v, acc, buf0, buf1, rs0, rs1, ss0, ss1, base)
    plsc.subcore_barrier()
    pltpu.sync_copy(acc.at[sl], out_hbm.at[cid, 1, sl])


def _sc_scatter(g, ii3, jj3, zeros_n):
    return pl.kernel(
        _sc_scatter_body,
        out_type=jax.ShapeDtypeStruct((NC, 2, NPAD, 128), _f32),
        mesh=_mesh(),
        scratch_types=[
            pltpu.VMEM((NCH, CH), jnp.int32),
            pltpu.VMEM((NCH, CH), jnp.int32),
            pltpu.VMEM((CH, 128), _f32),
            pltpu.VMEM((CH, 128), _f32),
            pltpu.VMEM_SHARED((NPAD, 128), _f32),
            pltpu.SemaphoreType.DMA,
            pltpu.SemaphoreType.DMA,
            pltpu.SemaphoreType.DMA,
            pltpu.SemaphoreType.DMA,
        ],
    )(g, ii3, jj3, zeros_n)


def _ln_relu(h):
    cnt = float(h.shape[0] * h.shape[1])
    m = jnp.sum(h) / cnt
    v = jnp.sum((h - m) * (h - m)) / cnt
    return jnp.maximum((h - m) * lax.rsqrt(v + EPS), 0.0)


def _node_open_body(xn_ref, w1t_ref, w2t_ref, out_ref):
    h = jnp.dot(xn_ref[...], w1t_ref[...], preferred_element_type=_f32)
    h = _ln_relu(h)
    r = jnp.dot(h, w2t_ref[...], preferred_element_type=_f32)
    out_ref[...] = jnp.concatenate([r, jnp.zeros_like(r)], axis=1)


def _node_open(xn_t, w1t, w2t):
    return pl.pallas_call(
        _node_open_body,
        out_shape=jax.ShapeDtypeStruct((N, 128), _f32),
    )(xn_t, w1t, w2t)


def _node_layer_body(parts_ref, xn_ref, wat_ref, wbt_ref, wct_ref, w2t_ref,
                     out_ref):
    p = parts_ref[...]
    s_i = p[0, 0, :N, :64] + p[1, 0, :N, :64]
    s_j = (p[0, 1, :N, :64] + p[1, 1, :N, :64]) - s_i
    ave = 0.5 * (s_i + s_j)
    div = s_i - s_j
    xn = xn_ref[...]
    h = jnp.dot(ave, wat_ref[...], preferred_element_type=_f32)
    h = h + jnp.dot(div, wbt_ref[...], preferred_element_type=_f32)
    h = h + jnp.dot(xn, wct_ref[...], preferred_element_type=_f32)
    h = _ln_relu(h)
    r = jnp.dot(h, w2t_ref[...], preferred_element_type=_f32)
    out_ref[...] = xn + HSTEP * jnp.concatenate(
        [r, jnp.zeros_like(r)], axis=1)


def _node_layer(parts, xn, wat, wbt, wct, w2t):
    return pl.pallas_call(
        _node_layer_body,
        out_shape=jax.ShapeDtypeStruct((N, 128), _f32),
    )(parts, xn, wat, wbt, wct, w2t)


def _final_body(xn_ref, wt_ref, out_ref):
    out_ref[...] = jnp.dot(xn_ref[...], wt_ref[...],
                           preferred_element_type=_f32)


def _final(xn, wt):
    return pl.pallas_call(
        _final_body,
        out_shape=jax.ShapeDtypeStruct((N, 128), _f32),
    )(xn, wt)


def _wspec(shape):
    return pl.BlockSpec(shape, lambda i: (0, 0))


def _eopen_stats_body(xe_ref, w1t_ref, sum_ref, sq_ref):
    i = pl.program_id(0)
    h = jnp.dot(xe_ref[...], w1t_ref[...], preferred_element_type=_f32)

    @pl.when(i == 0)
    def _():
        sum_ref[...] = jnp.zeros_like(sum_ref)
        sq_ref[...] = jnp.zeros_like(sq_ref)

    sum_ref[...] += jnp.sum(h, axis=0, keepdims=True)
    sq_ref[...] += jnp.sum(h * h, axis=0, keepdims=True)


def _eopen_stats(xe_t, w1t):
    return pl.pallas_call(
        _eopen_stats_body,
        grid=(NEB,),
        in_specs=[pl.BlockSpec((EB, 16), lambda i: (i, 0)), _wspec((16, 64))],
        out_specs=[pl.BlockSpec((1, 64), lambda i: (0, 0)),
                   pl.BlockSpec((1, 64), lambda i: (0, 0))],
        out_shape=[jax.ShapeDtypeStruct((1, 64), _f32),
                   jax.ShapeDtypeStruct((1, 64), _f32)],
    )(xe_t, w1t)


def _eopen_apply_body(xe_ref, w1t_ref, w2t_ref, sum_ref, sq_ref, out_ref):
    cnt = float(E * 64)
    m = jnp.sum(sum_ref[...]) / cnt
    v = jnp.sum(sq_ref[...]) / cnt - m * m
    h = jnp.dot(xe_ref[...], w1t_ref[...], preferred_element_type=_f32)
    h = jnp.maximum((h - m) * lax.rsqrt(v + EPS), 0.0)
    out_ref[...] = jnp.dot(h, w2t_ref[...], preferred_element_type=_f32)


def _eopen_apply(xe_t, w1t, w2t, sumv, sqv):
    return pl.pallas_call(
        _eopen_apply_body,
        grid=(NEB,),
        in_specs=[pl.BlockSpec((EB, 16), lambda i: (i, 0)),
                  _wspec((16, 64)), _wspec((64, 64)),
                  _wspec((1, 64)), _wspec((1, 64))],
        out_specs=pl.BlockSpec((EB, 64), lambda i: (i, 0)),
        out_shape=jax.ShapeDtypeStruct((E, 64), _f32),
    )(xe_t, w1t, w2t, sumv, sqv)


def _estats_body(xg_ref, xe_ref, wat_ref, wbt_ref, sum_ref, sq_ref):
    i = pl.program_id(0)
    h = jnp.dot(xg_ref[:, :64], wat_ref[...], preferred_element_type=_f32)
    h = h + jnp.dot(xe_ref[...], wbt_ref[...], preferred_element_type=_f32)

    @pl.when(i == 0)
    def _():
        sum_ref[...] = jnp.zeros_like(sum_ref)
        sq_ref[...] = jnp.zeros_like(sq_ref)

    sum_ref[...] += jnp.sum(h, axis=0, keepdims=True)
    sq_ref[...] += jnp.sum(h * h, axis=0, keepdims=True)


def _estats(xg, xe, wat, wbt):
    return pl.pallas_call(
        _estats_body,
        grid=(NEB,),
        in_specs=[pl.BlockSpec((EB, 128), lambda i: (i, 0)),
                  pl.BlockSpec((EB, 64), lambda i: (i, 0)),
                  _wspec((64, 128)), _wspec((64, 128))],
        out_specs=[pl.BlockSpec((1, 128), lambda i: (0, 0)),
                   pl.BlockSpec((1, 128), lambda i: (0, 0))],
        out_shape=[jax.ShapeDtypeStruct((1, 128), _f32),
                   jax.ShapeDtypeStruct((1, 128), _f32)],
    )(xg, xe, wat, wbt)


def _eapply_body(xg_ref, xe_ref, wat_ref, wbt_ref, w2t_ref, sum_ref, sq_ref,
                 g_ref, xeo_ref):
    cnt = float(E * 128)
    m = jnp.sum(sum_ref[...]) / cnt
    v = jnp.sum(sq_ref[...]) / cnt - m * m
    h = jnp.dot(xg_ref[:, :64], wat_ref[...], preferred_element_type=_f32)
    h = h + jnp.dot(xe_ref[...], wbt_ref[...], preferred_element_type=_f32)
    h = jnp.maximum((h - m) * lax.rsqrt(v + EPS), 0.0)
    g = jnp.dot(h, w2t_ref[...], preferred_element_type=_f32)
    g_ref[...] = jnp.concatenate([g, jnp.zeros_like(g)], axis=1)
    xeo_ref[...] = xe_ref[...] + HSTEP * g


def _eapply(xg, xe, wat, wbt, w2t, sumv, sqv):
    return pl.pallas_call(
        _eapply_body,
        grid=(NEB,),
        in_specs=[pl.BlockSpec((EB, 128), lambda i: (i, 0)),
                  pl.BlockSpec((EB, 64), lambda i: (i, 0)),
                  _wspec((64, 128)), _wspec((64, 128)), _wspec((128, 64)),
                  _wspec((1, 128)), _wspec((1, 128))],
        out_specs=[pl.BlockSpec((EB, 128), lambda i: (i, 0)),
                   pl.BlockSpec((EB, 64), lambda i: (i, 0))],
        out_shape=[jax.ShapeDtypeStruct((E, 128), _f32),
                   jax.ShapeDtypeStruct((E, 64), _f32)],
    )(xg, xe, wat, wbt, w2t, sumv, sqv)


def _pad_rows(w, rows):
    return jnp.concatenate([w, jnp.zeros((rows - w.shape[0], w.shape[1]),
                                         _f32)], axis=0)


def kernel(xn, xe, iInd, jInd, K1Nopen, K2Nopen, K1Eopen, K2Eopen, KNout,
           KE1, KE2, KN1, KN2):
    xn_t = jnp.transpose(xn[0])
    xe_t = jnp.transpose(xe[0])

    def _pad_idx(ind, fill):
        w = ind.astype(jnp.int32).reshape(NW, EPW)
        pad = jnp.full((NW, NCH * CH - EPW), fill, jnp.int32)
        return jnp.concatenate([w, pad], axis=1).reshape(NW, NCH, CH)

    gi3 = _pad_idx(iInd, 0)
    ii3 = _pad_idx(iInd, TRASH)
    jj3 = _pad_idx(jInd, TRASH)
    zeros_n = jnp.zeros((CH, 128), _f32)

    xn_c = _node_open(xn_t, K1Nopen.T, K2Nopen.T)
    xg = _sc_gather(xn_c, gi3)
    sumv, sqv = _eopen_stats(xe_t, K1Eopen.T)
    xe_c = _eopen_apply(xe_t, K1Eopen.T, K2Eopen.T, sumv, sqv)

    nlayers = KE1.shape[0]
    for i in range(nlayers):
        wat = jnp.transpose(KE1[i][:, :64] + KE1[i][:, 64:128])
        wbt = jnp.transpose(KE1[i][:, 128:])
        w2t = jnp.transpose(KE2[i])
        sumv, sqv = _estats(xg, xe_c, wat, wbt)
        g, xe_c = _eapply(xg, xe_c, wat, wbt, w2t, sumv, sqv)
        parts = _sc_scatter(g, ii3, jj3, zeros_n)
        xn_c = _node_layer(parts, xn_c,
                           jnp.transpose(KN1[i][:, :64]),
                           jnp.transpose(KN1[i][:, 64:128]),
                           _pad_rows(jnp.transpose(KN1[i][:, 128:]), 128),
                           jnp.transpose(KN2[i]))
        if i + 1 < nlayers:
            xg = _sc_gather(xn_c, gi3)

    xn_out = _final(xn_c, _pad_rows(KNout.T, 128))
    return jnp.transpose(xn_out)[None], jnp.transpose(xe_c[:, :64])[None]


# speedup vs baseline: 2.7946x; 1.0483x over previous
import jax
import jax.numpy as jnp
from jax import lax
from jax.experimental import pallas as pl
from jax.experimental.pallas import tpu as pltpu
from jax.experimental.pallas import tpu_sc as plsc

N = 10000
E = 160000
EPS = 1e-5
HSTEP = 0.1

NC = 2
NS = 16
NW = NC * NS
EPW = E // NW
CH = 120
NFULL = EPW // CH
TAIL = EPW - NFULL * CH
NCH = NFULL + 1
NPAD = 10240
NPS = NPAD // NS
TRASH = NPAD - 1

EB = 2000
NEB = E // EB

_f32 = jnp.float32


def _mesh():
    return plsc.VectorSubcoreMesh(core_axis_name="c", subcore_axis_name="s",
                                  num_cores=NC, num_subcores=NS)


def _sc_gather_body(table_hbm, idx_hbm, out_hbm, idx_v,
                    buf0, buf1, buf2, buf3,
                    gs0, gs1, gs2, gs3, ws0, ws1, ws2, ws3):
    cid = lax.axis_index("c")
    sid = lax.axis_index("s")
    wid = sid * NC + cid
    base = wid * EPW
    pltpu.sync_copy(idx_hbm.at[wid], idx_v)
    bufs = (buf0, buf1, buf2, buf3)
    gss = (gs0, gs1, gs2, gs3)
    wss = (ws0, ws1, ws2, ws3)

    def step(t, carry):
        hs = [pltpu.async_copy(table_hbm.at[idx_v.at[4 * t + k]],
                               bufs[k], gss[k]) for k in range(4)]
        ws = []
        for k in range(4):
            hs[k].wait()
            ws.append(pltpu.async_copy(
                bufs[k], out_hbm.at[pl.ds(base + (4 * t + k) * CH, CH)],
                wss[k]))
        for w in ws:
            w.wait()
        return carry

    lax.fori_loop(0, NFULL // 4, step, 0)
    rem = NFULL - (NFULL // 4) * 4
    hs = [pltpu.async_copy(table_hbm.at[idx_v.at[NFULL - rem + k]],
                           bufs[k], gss[k]) for k in range(rem)]
    ht = pltpu.async_copy(table_hbm.at[idx_v.at[NFULL]], bufs[rem], gss[rem])
    ws = []
    for k in range(rem):
        hs[k].wait()
        ws.append(pltpu.async_copy(
            bufs[k], out_hbm.at[pl.ds(base + (NFULL - rem + k) * CH, CH)],
            wss[k]))
    ht.wait()
    ws.append(pltpu.async_copy(bufs[rem].at[pl.ds(0, TAIL)],
                               out_hbm.at[pl.ds(base + NFULL * CH, TAIL)],
                               wss[rem]))
    for w in ws:
        w.wait()


def _sc_gather(table, idx3):
    return pl.kernel(
        _sc_gather_body,
        out_type=jax.ShapeDtypeStruct((E, 128), _f32),
        mesh=_mesh(),
        scratch_types=[
            pltpu.VMEM((NCH, CH), jnp.int32),
            pltpu.VMEM((CH, 128), _f32),
            pltpu.VMEM((CH, 128), _f32),
            pltpu.VMEM((CH, 128), _f32),
            pltpu.VMEM((CH, 128), _f32),
            pltpu.SemaphoreType.DMA,
            pltpu.SemaphoreType.DMA,
            pltpu.SemaphoreType.DMA,
            pltpu.SemaphoreType.DMA,
            pltpu.SemaphoreType.DMA,
            pltpu.SemaphoreType.DMA,
            pltpu.SemaphoreType.DMA,
            pltpu.SemaphoreType.DMA,
        ],
    )(table, idx3)


def _scatter_phase(g_hbm, idx_v, acc, buf0, buf1, rs0, rs1, ss0, ss1, base):
    def step(t, carry):
        c0 = 2 * t
        c1 = 2 * t + 1
        r0 = pltpu.async_copy(g_hbm.at[pl.ds(base + c0 * CH, CH)], buf0, rs0)
        r1 = pltpu.async_copy(g_hbm.at[pl.ds(base + c1 * CH, CH)], buf1, rs1)
        r0.wait()
        s0 = pltpu.async_copy(buf0, acc.at[idx_v.at[c0]], ss0, add=True)
        r1.wait()
        s1 = pltpu.async_copy(buf1, acc.at[idx_v.at[c1]], ss1, add=True)
        s0.wait()
        s1.wait()
        return carry

    lax.fori_loop(0, NFULL // 2, step, 0)
    c0 = NFULL - 1
    r0 = pltpu.async_copy(g_hbm.at[pl.ds(base + c0 * CH, CH)], buf0, rs0)
    r1 = pltpu.async_copy(g_hbm.at[pl.ds(base + NFULL * CH, TAIL)],
                          buf1.at[pl.ds(0, TAIL)], rs1)
    r0.wait()
    s0 = pltpu.async_copy(buf0, acc.at[idx_v.at[c0]], ss0, add=True)
    r1.wait()
    s1 = pltpu.async_copy(buf1, acc.at[idx_v.at[NFULL]], ss1, add=True)
    s0.wait()
    s1.wait()


def _sc_scatter_body(g_hbm, ii_hbm, jj_hbm, z_hbm, out_hbm,
                     ii_v, jj_v, buf0, buf1, acc,
                     rs0, rs1, ss0, ss1):
    cid = lax.axis_index("c")
    sid = lax.axis_index("s")
    wid = sid * NC + cid
    base = wid * EPW
    sl = pl.ds(sid * NPS, NPS)
    pltpu.sync_copy(ii_hbm.at[wid], ii_v)
    pltpu.sync_copy(jj_hbm.at[wid], jj_v)
    pltpu.sync_copy(z_hbm, buf0)
    for k in range(NPS // CH):
        pltpu.sync_copy(buf0, acc.at[pl.ds(sid * NPS + k * CH, CH)])
    rem = NPS - (NPS // CH) * CH
    if rem:
        pltpu.sync_copy(buf0.at[pl.ds(0, rem)],
                        acc.at[pl.ds(sid * NPS + (NPS // CH) * CH, rem)])
    plsc.subcore_barrier()
    _scatter_phase(g_hbm, ii_v, acc, buf0, buf1, rs0, rs1, ss0, ss1, base)
    plsc.subcore_barrier()
    pltpu.sync_copy(acc.at[sl], out_hbm.at[cid, 0, sl])
    plsc.subcore_barrier()
    _scatter_phase(g_hbm, jj_v, acc, buf0, buf1, rs0, rs1, ss0, ss1, base)
    plsc.subcore_barrier()
    pltpu.sync_copy(acc.at[sl], out_hbm.at[cid, 1, sl])


def _sc_scatter(g, ii3, jj3, zeros_n):
    return pl.kernel(
        _sc_scatter_body,
        out_type=jax.ShapeDtypeStruct((NC, 2, NPAD, 128), _f32),
        mesh=_mesh(),
        scratch_types=[
            pltpu.VMEM((NCH, CH), jnp.int32),
            pltpu.VMEM((NCH, CH), jnp.int32),
            pltpu.VMEM((CH, 128), _f32),
            pltpu.VMEM((CH, 128), _f32),
            pltpu.VMEM_SHARED((NPAD, 128), _f32),
            pltpu.SemaphoreType.DMA,
            pltpu.SemaphoreType.DMA,
            pltpu.SemaphoreType.DMA,
            pltpu.SemaphoreType.DMA,
        ],
    )(g, ii3, jj3, zeros_n)


def _ln_relu(h):
    cnt = float(h.shape[0] * h.shape[1])
    m = jnp.sum(h) / cnt
    v = jnp.sum((h - m) * (h - m)) / cnt
    return jnp.maximum((h - m) * lax.rsqrt(v + EPS), 0.0)


def _node_open_body(xn_ref, w1t_ref, w2t_ref, out_ref):
    h = jnp.dot(xn_ref[...], w1t_ref[...], preferred_element_type=_f32)
    h = _ln_relu(h)
    r = jnp.dot(h, w2t_ref[...], preferred_element_type=_f32)
    out_ref[...] = jnp.concatenate([r, jnp.zeros_like(r)], axis=1)


def _node_open(xn_t, w1t, w2t):
    return pl.pallas_call(
        _node_open_body,
        out_shape=jax.ShapeDtypeStruct((N, 128), _f32),
    )(xn_t, w1t, w2t)


def _node_layer_body(parts_ref, xn_ref, wat_ref, wbt_ref, wct_ref, w2t_ref,
                     out_ref):
    p = parts_ref[...]
    s_i = p[0, 0, :N, :64] + p[1, 0, :N, :64]
    s_j = (p[0, 1, :N, :64] + p[1, 1, :N, :64]) - s_i
    ave = 0.5 * (s_i + s_j)
    div = s_i - s_j
    xn = xn_ref[...]
    h = jnp.dot(ave, wat_ref[...], preferred_element_type=_f32)
    h = h + jnp.dot(div, wbt_ref[...], preferred_element_type=_f32)
    h = h + jnp.dot(xn, wct_ref[...], preferred_element_type=_f32)
    h = _ln_relu(h)
    r = jnp.dot(h, w2t_ref[...], preferred_element_type=_f32)
    out_ref[...] = xn + HSTEP * jnp.concatenate(
        [r, jnp.zeros_like(r)], axis=1)


def _node_layer(parts, xn, wat, wbt, wct, w2t):
    return pl.pallas_call(
        _node_layer_body,
        out_shape=jax.ShapeDtypeStruct((N, 128), _f32),
    )(parts, xn, wat, wbt, wct, w2t)


def _final_body(xn_ref, wt_ref, out_ref):
    out_ref[...] = jnp.dot(xn_ref[...], wt_ref[...],
                           preferred_element_type=_f32)


def _final(xn, wt):
    return pl.pallas_call(
        _final_body,
        out_shape=jax.ShapeDtypeStruct((N, 128), _f32),
    )(xn, wt)


def _wspec(shape):
    return pl.BlockSpec(shape, lambda i: (0, 0))


def _eopen_stats_body(xe_ref, w1t_ref, sum_ref, sq_ref):
    i = pl.program_id(0)
    h = jnp.dot(xe_ref[...], w1t_ref[...], preferred_element_type=_f32)

    @pl.when(i == 0)
    def _():
        sum_ref[...] = jnp.zeros_like(sum_ref)
        sq_ref[...] = jnp.zeros_like(sq_ref)

    sum_ref[...] += jnp.sum(h, axis=0, keepdims=True)
    sq_ref[...] += jnp.sum(h * h, axis=0, keepdims=True)


def _eopen_stats(xe_t, w1t):
    return pl.pallas_call(
        _eopen_stats_body,
        grid=(NEB,),
        in_specs=[pl.BlockSpec((EB, 16), lambda i: (i, 0)), _wspec((16, 64))],
        out_specs=[pl.BlockSpec((1, 64), lambda i: (0, 0)),
                   pl.BlockSpec((1, 64), lambda i: (0, 0))],
        out_shape=[jax.ShapeDtypeStruct((1, 64), _f32),
                   jax.ShapeDtypeStruct((1, 64), _f32)],
    )(xe_t, w1t)


def _eopen_apply_body(xe_ref, w1t_ref, w2t_ref, sum_ref, sq_ref, out_ref):
    cnt = float(E * 64)
    m = jnp.sum(sum_ref[...]) / cnt
    v = jnp.sum(sq_ref[...]) / cnt - m * m
    h = jnp.dot(xe_ref[...], w1t_ref[...], preferred_element_type=_f32)
    h = jnp.maximum((h - m) * lax.rsqrt(v + EPS), 0.0)
    out_ref[...] = jnp.dot(h, w2t_ref[...], preferred_element_type=_f32)


def _eopen_apply(xe_t, w1t, w2t, sumv, sqv):
    return pl.pallas_call(
        _eopen_apply_body,
        grid=(NEB,),
        in_specs=[pl.BlockSpec((EB, 16), lambda i: (i, 0)),
                  _wspec((16, 64)), _wspec((64, 64)),
                  _wspec((1, 64)), _wspec((1, 64))],
        out_specs=pl.BlockSpec((EB, 64), lambda i: (i, 0)),
        out_shape=jax.ShapeDtypeStruct((E, 64), _f32),
    )(xe_t, w1t, w2t, sumv, sqv)


def _elayer_body(xg_ref, xe_ref, wat_ref, wbt_ref, w2t_ref,
                 g_ref, xeo_ref, h_scr, sum_ref, sq_ref):
    i2 = pl.program_id(0)

    @pl.when(i2 == 0)
    def _():
        sum_ref[...] = jnp.zeros_like(sum_ref)
        sq_ref[...] = jnp.zeros_like(sq_ref)

    @pl.when(i2 < NEB)
    def _():
        h = jnp.dot(xg_ref[:, :64], wat_ref[...],
                    preferred_element_type=_f32)
        h = h + jnp.dot(xe_ref[...], wbt_ref[...],
                        preferred_element_type=_f32)
        h_scr[pl.ds(i2 * EB, EB), :] = h.astype(jnp.bfloat16)
        sum_ref[...] += jnp.sum(h, axis=0, keepdims=True)
        sq_ref[...] += jnp.sum(h * h, axis=0, keepdims=True)

    @pl.when(i2 >= NEB)
    def _():
        i = i2 - NEB
        cnt = float(E * 128)
        m = jnp.sum(sum_ref[...]) / cnt
        v = jnp.sum(sq_ref[...]) / cnt - m * m
        h = h_scr[pl.ds(i * EB, EB), :].astype(_f32)
        hn = jnp.maximum((h - m) * lax.rsqrt(v + EPS), 0.0)
        g = jnp.dot(hn, w2t_ref[...], preferred_element_type=_f32)
        g_ref[...] = jnp.concatenate([g, jnp.zeros_like(g)], axis=1)
        xeo_ref[...] = xe_ref[...] + HSTEP * g


def _elayer(xg, xe, wat, wbt, w2t):
    return pl.pallas_call(
        _elayer_body,
        grid=(2 * NEB,),
        in_specs=[
            pl.BlockSpec((EB, 128),
                         lambda i2: (jnp.where(i2 < NEB, i2, 0), 0)),
            pl.BlockSpec((EB, 64),
                         lambda i2: (jnp.where(i2 < NEB, i2, i2 - NEB), 0)),
            _wspec((64, 128)), _wspec((64, 128)), _wspec((128, 64)),
        ],
        out_specs=[
            pl.BlockSpec((EB, 128),
                         lambda i2: (jnp.where(i2 < NEB, 0, i2 - NEB), 0)),
            pl.BlockSpec((EB, 64),
                         lambda i2: (jnp.where(i2 < NEB, 0, i2 - NEB), 0)),
        ],
        out_shape=[jax.ShapeDtypeStruct((E, 128), _f32),
                   jax.ShapeDtypeStruct((E, 64), _f32)],
        scratch_shapes=[pltpu.VMEM((E, 128), jnp.bfloat16),
                        pltpu.VMEM((1, 128), _f32),
                        pltpu.VMEM((1, 128), _f32)],
        compiler_params=pltpu.CompilerParams(
            vmem_limit_bytes=64 * 1024 * 1024),
    )(xg, xe, wat, wbt, w2t)


def _pad_rows(w, rows):
    return jnp.concatenate([w, jnp.zeros((rows - w.shape[0], w.shape[1]),
                                         _f32)], axis=0)


def kernel(xn, xe, iInd, jInd, K1Nopen, K2Nopen, K1Eopen, K2Eopen, KNout,
           KE1, KE2, KN1, KN2):
    xn_t = jnp.transpose(xn[0])
    xe_t = jnp.transpose(xe[0])

    def _pad_idx(ind, fill):
        w = ind.astype(jnp.int32).reshape(NW, EPW)
        pad = jnp.full((NW, NCH * CH - EPW), fill, jnp.int32)
        return jnp.concatenate([w, pad], axis=1).reshape(NW, NCH, CH)

    gi3 = _pad_idx(iInd, 0)
    ii3 = _pad_idx(iInd, TRASH)
    jj3 = _pad_idx(jInd, TRASH)
    zeros_n = jnp.zeros((CH, 128), _f32)

    xn_c = _node_open(xn_t, K1Nopen.T, K2Nopen.T)
    xg = _sc_gather(xn_c, gi3)
    sumv, sqv = _eopen_stats(xe_t, K1Eopen.T)
    xe_c = _eopen_apply(xe_t, K1Eopen.T, K2Eopen.T, sumv, sqv)

    nlayers = KE1.shape[0]
    for i in range(nlayers):
        wat = jnp.transpose(KE1[i][:, :64] + KE1[i][:, 64:128])
        wbt = jnp.transpose(KE1[i][:, 128:])
        w2t = jnp.transpose(KE2[i])
        g, xe_c = _elayer(xg, xe_c, wat, wbt, w2t)
        parts = _sc_scatter(g, ii3, jj3, zeros_n)
        xn_c = _node_layer(parts, xn_c,
                           jnp.transpose(KN1[i][:, :64]),
                           jnp.transpose(KN1[i][:, 64:128]),
                           _pad_rows(jnp.transpose(KN1[i][:, 128:]), 128),
                           jnp.transpose(KN2[i]))
        if i + 1 < nlayers:
            xg = _sc_gather(xn_c, gi3)

    xn_out = _final(xn_c, _pad_rows(KNout.T, 128))
    return jnp.transpose(xn_out)[None], jnp.transpose(xe_c[:, :64])[None]
